# baseline (speedup 1.0000x reference)
import functools

import jax
import jax.numpy as jnp
from jax import lax
from jax.experimental import pallas as pl
from jax.experimental.pallas import tpu as pltpu
from jax.experimental.pallas import tpu_sc as plsc

D = 32
G = 128
NW = 32
B_TOTAL = 4096 * 200
B_PER_W = B_TOTAL // NW
NU = B_PER_W // G
NT = 4096 // 128

_mesh = plsc.VectorSubcoreMesh(core_axis_name="c", subcore_axis_name="s")


@functools.partial(
    pl.kernel,
    out_type=jax.ShapeDtypeStruct((200, 4, NT, 8, 128), jnp.float32),
    mesh=_mesh,
    scratch_types=[
        pltpu.VMEM((B_PER_W,), jnp.int32),
        pltpu.VMEM((2, G, D), jnp.float32),
        pltpu.VMEM((2, D, G), jnp.float32),
        pltpu.SemaphoreType.DMA,
        pltpu.SemaphoreType.DMA,
        pltpu.SemaphoreType.DMA,
        pltpu.SemaphoreType.DMA,
    ],
    compiler_params=pltpu.CompilerParams(
        use_tc_tiling_on_sc=False,
        needs_layout_passes=False,
        disable_bounds_checks=True,
    ),
)
def _emb_lookup(xs_hbm, table_hbm, out_hbm, idx_v, rows_v, t_v, g0, g1, w0, w1):
    wid = lax.axis_index("s") * 2 + lax.axis_index("c")
    u_base = wid * NU
    gsem = (g0, g1)
    wsem = (w0, w1)
    iota = lax.iota(jnp.int32, 16)

    pltpu.sync_copy(xs_hbm.at[pl.ds(u_base * G, B_PER_W)], idx_v)

    def fire_gather(ci, p):
        pltpu.async_copy(
            table_hbm.at[idx_v.at[pl.ds(ci * G, G)]], rows_v.at[p], gsem[p]
        )

    def wait_gather(p):
        pltpu.make_async_copy(
            table_hbm.at[idx_v.at[pl.ds(0, G)]], rows_v.at[p], gsem[p]
        ).wait()

    rowsel = [iota + (b0 * 16) for b0 in range(G // 16)]
    colsel = [jnp.full((16,), d, jnp.int32) for d in range(D)]

    def transpose(p):
        rows = rows_v.at[p]
        dst = t_v.at[p]
        for d in range(D):
            for b0 in range(G // 16):
                vec = plsc.load_gather(rows, [rowsel[b0], colsel[d]])
                dst[d, pl.ds(b0 * 16, 16)] = vec

    def fire_writes(ci, p):
        u = u_base + ci
        t = u // NT
        bc = lax.rem(u, NT)
        for tr in range(4):
            pltpu.async_copy(
                t_v.at[p].at[pl.ds(tr * 8, 8)], out_hbm.at[t, tr, bc], wsem[p]
            )

    def wait_writes(p):
        for tr in range(4):
            pltpu.make_async_copy(
                t_v.at[p].at[pl.ds(tr * 8, 8)], out_hbm.at[0, tr, 0], wsem[p]
            ).wait()

    fire_gather(0, 0)

    @pl.loop(0, NU, step=2)
    def _step(ci):
        fire_gather(ci + 1, 1)
        wait_gather(0)
        @pl.when(ci > 0)
        def _():
            wait_writes(0)
        transpose(0)
        fire_writes(ci, 0)
        @pl.when(ci + 2 < NU)
        def _():
            fire_gather(ci + 2, 0)
        wait_gather(1)
        @pl.when(ci > 0)
        def _():
            wait_writes(1)
        transpose(1)
        fire_writes(ci + 1, 1)

    wait_writes(0)
    wait_writes(1)


def kernel(xs, table):
    out5 = _emb_lookup(xs.T.reshape(B_TOTAL), table)
    return out5.transpose(2, 4, 0, 1, 3).reshape(4096, 200, D)


# speedup vs baseline: 1.4385x; 1.4385x over previous
import functools

import jax
import jax.numpy as jnp
from jax import lax
from jax.experimental import pallas as pl
from jax.experimental.pallas import tpu as pltpu
from jax.experimental.pallas import tpu_sc as plsc

D = 32
G = 128
NW = 32
B_TOTAL = 4096 * 200
B_PER_W = B_TOTAL // NW
NU = B_PER_W // G
NT = 4096 // 128

_mesh = plsc.VectorSubcoreMesh(core_axis_name="c", subcore_axis_name="s")


@functools.partial(
    pl.kernel,
    out_type=jax.ShapeDtypeStruct((200, 4, NT, 8, 128), jnp.float32),
    mesh=_mesh,
    scratch_types=[
        pltpu.VMEM((B_PER_W,), jnp.int32),
        pltpu.VMEM((2, G, D), jnp.float32),
        pltpu.VMEM((2, D, G), jnp.float32),
        pltpu.SemaphoreType.DMA,
        pltpu.SemaphoreType.DMA,
        pltpu.SemaphoreType.DMA,
        pltpu.SemaphoreType.DMA,
    ],
    compiler_params=pltpu.CompilerParams(
        use_tc_tiling_on_sc=False,
        needs_layout_passes=False,
        disable_bounds_checks=True,
    ),
)
def _emb_lookup(xs_hbm, table_hbm, out_hbm, idx_v, rows_v, t_v, g0, g1, w0, w1):
    wid = lax.axis_index("s") * 2 + lax.axis_index("c")
    u_base = wid * NU
    gsem = (g0, g1)
    wsem = (w0, w1)
    iota = lax.iota(jnp.int32, 16)

    pltpu.sync_copy(xs_hbm.at[pl.ds(u_base * G, B_PER_W)], idx_v)

    def fire_gather(ci, p):
        pltpu.async_copy(
            table_hbm.at[idx_v.at[pl.ds(ci * G, G)]], rows_v.at[p], gsem[p]
        )

    def wait_gather(p):
        pltpu.make_async_copy(
            table_hbm.at[idx_v.at[pl.ds(0, G)]], rows_v.at[p], gsem[p]
        ).wait()

    rowsel = [iota + (b0 * 16) for b0 in range(G // 16)]

    def transpose(p):
        rows = rows_v.at[p]
        dst = t_v.at[p]

        @plsc.parallel_loop(0, D, unroll=4)
        def _d(d):
            col = jnp.full((16,), d, jnp.int32)
            for b0 in range(G // 16):
                vec = plsc.load_gather(rows, [rowsel[b0], col])
                dst[d, pl.ds(b0 * 16, 16)] = vec

    def fire_writes(ci, p):
        u = u_base + ci
        t = u // NT
        bc = lax.rem(u, NT)
        for tr in range(4):
            pltpu.async_copy(
                t_v.at[p].at[pl.ds(tr * 8, 8)], out_hbm.at[t, tr, bc], wsem[p]
            )

    def wait_writes(p):
        for tr in range(4):
            pltpu.make_async_copy(
                t_v.at[p].at[pl.ds(tr * 8, 8)], out_hbm.at[0, tr, 0], wsem[p]
            ).wait()

    fire_gather(0, 0)

    @pl.loop(0, NU, step=2)
    def _step(ci):
        fire_gather(ci + 1, 1)
        wait_gather(0)
        @pl.when(ci > 0)
        def _():
            wait_writes(0)
        transpose(0)
        fire_writes(ci, 0)
        @pl.when(ci + 2 < NU)
        def _():
            fire_gather(ci + 2, 0)
        wait_gather(1)
        @pl.when(ci > 0)
        def _():
            wait_writes(1)
        transpose(1)
        fire_writes(ci + 1, 1)

    wait_writes(0)
    wait_writes(1)


def kernel(xs, table):
    out5 = _emb_lookup(xs.T.reshape(B_TOTAL), table)
    return out5.transpose(2, 4, 0, 1, 3).reshape(4096, 200, D)
